# baseline (speedup 1.0000x reference)
import functools

import jax
import jax.numpy as jnp
from jax import lax
from jax.experimental import pallas as pl
from jax.experimental.pallas import tpu as pltpu
from jax.experimental.pallas import tpu_sc as plsc

LANES = 16


def _make_sc_kernel(B, S, D, dtype):
    info = plsc.get_sparse_core_info()
    NC, NS = info.num_cores, info.num_subcores
    NW = NC * NS
    assert S % NW == 0
    s_per_w = S // NW
    CH = 32
    assert s_per_w % CH == 0
    n_chunks = s_per_w // CH
    nvec = D // LANES
    mesh = plsc.VectorSubcoreMesh(core_axis_name="c", subcore_axis_name="s")

    @functools.partial(
        pl.kernel,
        mesh=mesh,
        out_type=jax.ShapeDtypeStruct((B, S, D), dtype),
        scratch_types=[
            pltpu.VMEM((CH, D), dtype),
            pltpu.VMEM((CH, D), dtype),
        ],
    )
    def k(we_hbm, p_hbm, out_hbm, p_buf, we_buf):
        wid = lax.axis_index("s") * NC + lax.axis_index("c")
        base = wid * s_per_w

        def chunk_body(kk, _):
            s0 = base + kk * CH
            pltpu.sync_copy(p_hbm.at[pl.ds(s0, CH)], p_buf)
            for b in range(B):
                pltpu.sync_copy(we_hbm.at[b, pl.ds(s0, CH)], we_buf)

                def row_body(r, _):
                    for cv in range(nvec):
                        sl = pl.ds(cv * LANES, LANES)
                        we_buf[r, sl] = we_buf[r, sl] + p_buf[r, sl]
                    return 0

                lax.fori_loop(0, CH, row_body, 0)
                pltpu.sync_copy(we_buf, out_hbm.at[b, pl.ds(s0, CH)])
            return 0

        lax.fori_loop(0, n_chunks, chunk_body, 0)

    return k


def kernel(inputs, word_embeddings, P):
    del inputs
    B, S, D = word_embeddings.shape
    if P.shape[0] != S:
        P = P[:S]
    k = _make_sc_kernel(B, S, D, word_embeddings.dtype)
    return k(word_embeddings, P)


# speedup vs baseline: 1.0021x; 1.0021x over previous
import functools

import jax
import jax.numpy as jnp
from jax import lax
from jax.experimental import pallas as pl
from jax.experimental.pallas import tpu as pltpu
from jax.experimental.pallas import tpu_sc as plsc

LANES = 16
CH = 16


def _make_sc_kernel(B, S, D, dtype):
    info = plsc.get_sparse_core_info()
    NC, NS = info.num_cores, info.num_subcores
    NW = NC * NS
    assert S % NW == 0
    s_per_w = S // NW
    assert s_per_w % (2 * CH) == 0
    n_chunks = s_per_w // CH
    nvec = D // LANES
    mesh = plsc.VectorSubcoreMesh(core_axis_name="c", subcore_axis_name="s")

    @functools.partial(
        pl.kernel,
        mesh=mesh,
        out_type=jax.ShapeDtypeStruct((B, S, D), dtype),
        scratch_types=[
            pltpu.VMEM((B, CH, D), dtype),
            pltpu.VMEM((B, CH, D), dtype),
            pltpu.VMEM((CH, D), dtype),
            pltpu.VMEM((CH, D), dtype),
            pltpu.SemaphoreType.DMA,
            pltpu.SemaphoreType.DMA,
            pltpu.SemaphoreType.DMA,
            pltpu.SemaphoreType.DMA,
            pltpu.SemaphoreType.DMA,
            pltpu.SemaphoreType.DMA,
        ],
    )
    def k(we_hbm, p_hbm, out_hbm, wb0, wb1, pb0, pb1,
          sw0, sw1, sp0, sp1, so0, so1):
        wid = lax.axis_index("s") * NC + lax.axis_index("c")
        base = wid * s_per_w
        wb = [wb0, wb1]
        pb = [pb0, pb1]
        sw = [sw0, sw1]
        sp = [sp0, sp1]
        so = [so0, so1]

        def in_copies(c, j):
            s0 = base + c * CH
            return (
                pltpu.make_async_copy(we_hbm.at[:, pl.ds(s0, CH)], wb[j], sw[j]),
                pltpu.make_async_copy(p_hbm.at[pl.ds(s0, CH)], pb[j], sp[j]),
            )

        def out_copy(c, j):
            s0 = base + c * CH
            return pltpu.make_async_copy(wb[j], out_hbm.at[:, pl.ds(s0, CH)], so[j])

        for cp in in_copies(0, 0):
            cp.start()

        def step(c, j):
            @pl.when(c >= 1)
            def _():
                out_copy(c - 1, 1 - j).wait()

            @pl.when(c + 1 < n_chunks)
            def _():
                for cp in in_copies(c + 1, 1 - j):
                    cp.start()

            for cp in in_copies(c, j):
                cp.wait()

            def row_body(r, _):
                for cv in range(nvec):
                    sl = pl.ds(cv * LANES, LANES)
                    pv = pb[j][r, sl]
                    for b in range(B):
                        wb[j][b, r, sl] = wb[j][b, r, sl] + pv
                return 0

            lax.fori_loop(0, CH, row_body, 0)
            out_copy(c, j).start()

        def pair_body(g, _):
            step(2 * g, 0)
            step(2 * g + 1, 1)
            return 0

        lax.fori_loop(0, n_chunks // 2, pair_body, 0)
        out_copy(n_chunks - 1, 1).wait()

    return k


def kernel(inputs, word_embeddings, P):
    del inputs
    B, S, D = word_embeddings.shape
    if P.shape[0] != S:
        P = P[:S]
    k = _make_sc_kernel(B, S, D, word_embeddings.dtype)
    return k(word_embeddings, P)


# speedup vs baseline: 2.4950x; 2.4898x over previous
import jax
import jax.numpy as jnp
from jax.experimental import pallas as pl

SBLK = 256


def _add_body(we_ref, p_ref, out_ref):
    out_ref[...] = we_ref[...] + p_ref[...][None, :, :]


def kernel(inputs, word_embeddings, P):
    del inputs
    B, S, D = word_embeddings.shape
    if P.shape[0] != S:
        P = P[:S]
    grid = (S // SBLK,)
    return pl.pallas_call(
        _add_body,
        grid=grid,
        in_specs=[
            pl.BlockSpec((B, SBLK, D), lambda i: (0, i, 0)),
            pl.BlockSpec((SBLK, D), lambda i: (i, 0)),
        ],
        out_specs=pl.BlockSpec((B, SBLK, D), lambda i: (0, i, 0)),
        out_shape=jax.ShapeDtypeStruct((B, S, D), word_embeddings.dtype),
    )(word_embeddings, P)


# speedup vs baseline: 2.5594x; 1.0258x over previous
import jax
import jax.numpy as jnp
from jax.experimental import pallas as pl

SBLK = 1024


def _add_body(we_ref, p_ref, out_ref):
    out_ref[...] = we_ref[...] + p_ref[...][None, :, :]


def kernel(inputs, word_embeddings, P):
    del inputs
    B, S, D = word_embeddings.shape
    if P.shape[0] != S:
        P = P[:S]
    grid = (S // SBLK,)
    return pl.pallas_call(
        _add_body,
        grid=grid,
        in_specs=[
            pl.BlockSpec((B, SBLK, D), lambda i: (0, i, 0)),
            pl.BlockSpec((SBLK, D), lambda i: (i, 0)),
        ],
        out_specs=pl.BlockSpec((B, SBLK, D), lambda i: (0, i, 0)),
        out_shape=jax.ShapeDtypeStruct((B, S, D), word_embeddings.dtype),
    )(word_embeddings, P)
